# Initial kernel scaffold; baseline (speedup 1.0000x reference)
import jax
import jax.numpy as jnp
from jax.experimental import pallas as pl


def kernel(seq_len, bias_table):
    raise NotImplementedError("write your pallas kernel here")


# speedup vs baseline: 22.5935x; 22.5935x over previous
import functools

import jax
import jax.numpy as jnp
from jax import lax
from jax.experimental import pallas as pl
from jax.experimental.pallas import tpu as pltpu
from jax.experimental.pallas import tpu_sc as plsc

NUM_HEADS = 32
NUM_BUCKETS = 32
SEQ = 2048
L = 16

_THRESH = (19, 21, 24, 27, 31, 35, 40, 46, 52, 59, 67, 77, 87, 99, 113)

NSHIFT = 8
GLEN = 2 * SEQ + L
FCOLS = 2 * SEQ
RPB = 8


def _body(table_hbm, out_hbm, tab_v, g_v, f_v, sem):
    nc = 2
    h = lax.axis_index("s") * nc + lax.axis_index("c")
    pltpu.sync_copy(table_hbm, tab_v)
    iota = lax.iota(jnp.int32, L)
    h_vec = jnp.full((L,), h, dtype=jnp.int32)
    ones = jnp.full((L,), 1, dtype=jnp.int32)
    zeros = jnp.full((L,), 0, dtype=jnp.int32)

    def build_g(a, carry):
        t = a * L + iota
        d = jnp.maximum((SEQ - 1) - t, 0)
        acc = jnp.full((L,), 16, dtype=jnp.int32)
        for thr in _THRESH:
            acc = acc + jnp.where(d >= thr, ones, zeros)
        bucket = jnp.where(d < 16, d, acc)
        g_v[pl.ds(a * L, L)] = plsc.load_gather(
            tab_v, [bucket * NUM_HEADS + h_vec])
        return carry

    lax.fori_loop(0, GLEN // L, build_g, None)

    def build_f(a, carry):
        base = a * L + iota
        for s in range(NSHIFT):
            f_v[pl.ds(s * FCOLS + a * L, L)] = plsc.load_gather(
                g_v, [base + s])
        return carry

    lax.fori_loop(0, FCOLS // L, build_f, None)

    def send(blk, carry):
        i0 = pl.multiple_of(blk * RPB, 8)
        cbase = (SEQ - NSHIFT) - i0
        copies = []
        for r in range(RPB):
            s = NSHIFT - 1 - r
            copies.append(pltpu.async_copy(
                f_v.at[pl.ds(s * FCOLS + cbase, SEQ)],
                out_hbm.at[h, i0 + r, :], sem))
        for cp in copies:
            cp.wait()
        return carry

    lax.fori_loop(0, SEQ // RPB, send, None)


def kernel(seq_len, bias_table):
    del seq_len
    run = functools.partial(
        pl.kernel,
        mesh=plsc.VectorSubcoreMesh(core_axis_name="c", subcore_axis_name="s"),
        compiler_params=pltpu.CompilerParams(
            needs_layout_passes=False, use_tc_tiling_on_sc=False),
        out_type=jax.ShapeDtypeStruct((NUM_HEADS, SEQ, SEQ), jnp.float32),
        scratch_types=[
            pltpu.VMEM((NUM_BUCKETS * NUM_HEADS,), jnp.float32),
            pltpu.VMEM((GLEN,), jnp.float32),
            pltpu.VMEM((NSHIFT * FCOLS,), jnp.float32),
            pltpu.SemaphoreType.DMA,
        ],
    )(_body)
    return run(bias_table.reshape(-1))[None]
